# Initial kernel scaffold; baseline (speedup 1.0000x reference)
import functools

import jax
import jax.numpy as jnp
from jax import lax
from jax.experimental import pallas as pl
from jax.experimental.pallas import tpu as pltpu
from jax.experimental.pallas import tpu_sc as plsc

N0, N1, N2 = 10000, 5000, 2500
E0, E1 = 160000, 80000
H = 256

N0P, N1P, N2P = 10240, 5120, 2560
E0P, E1P = 163840, 81920

_MESH = plsc.VectorSubcoreMesh(core_axis_name="c", subcore_axis_name="s")


def _make_sc_conv(Ncp, Nfp, Ep, identity_cluster):
    per_tile = Ep // 16
    nblk = per_tile // 128
    rpt = Nfp // 16
    RC = 128 if rpt % 128 == 0 else 64
    nrc = rpt // RC

    scratch = []
    if not identity_cluster:
        scratch.append(pltpu.VMEM((Nfp,), jnp.int32))
    scratch += [
        pltpu.VMEM((per_tile,), jnp.int32),
        pltpu.VMEM((per_tile,), jnp.int32),
        pltpu.VMEM((128,), jnp.int32),
        pltpu.VMEM((128,), jnp.int32),
        pltpu.VMEM((RC,), jnp.int32),
        pltpu.VMEM((128, 128), jnp.float32),
        pltpu.VMEM((RC, 128), jnp.float32),
        pltpu.VMEM_SHARED((Nfp, 128), jnp.float32),
        pltpu.SemaphoreType.DMA,
    ]

    def body(*refs):
        if identity_cluster:
            (src_h, dst_h, rel_h, root_h, out_h,
             src_v, dst_v, comp, dstb, compi, rows_v, rows_i, acc, sem) = refs
            cluster_v = None
        else:
            (src_h, dst_h, cluster_h, rel_h, root_h, out_h,
             cluster_v, src_v, dst_v, comp, dstb, compi, rows_v, rows_i,
             acc, sem) = refs

        c = lax.axis_index("c")
        t = lax.axis_index("s")
        c_off = (c * Ncp).astype(jnp.int32)

        pltpu.sync_copy(src_h.at[t], src_v)
        pltpu.sync_copy(dst_h.at[t], dst_v)
        if not identity_cluster:
            pltpu.sync_copy(cluster_h, cluster_v)

        for q in range(nrc):
            r0 = t * rpt + q * RC
            if identity_cluster:
                pltpu.sync_copy(root_h.at[pl.ds(c * Nfp + r0, RC)], rows_i)
            else:
                for k in range(RC // 16):
                    cv = cluster_v[pl.ds(r0 + k * 16, 16)]
                    compi[pl.ds(k * 16, 16)] = cv + c_off
                pltpu.async_copy(root_h.at[compi], rows_i, sem).wait()
            pltpu.sync_copy(rows_i, acc.at[pl.ds(r0, RC)])

        plsc.subcore_barrier()

        def blk(j, carry):
            base = j * 128
            for k in range(8):
                s16 = src_v[pl.ds(base + k * 16, 16)]
                if identity_cluster:
                    cg = s16
                else:
                    cg = plsc.load_gather(cluster_v, [s16])
                comp[pl.ds(k * 16, 16)] = cg + c_off
                dstb[pl.ds(k * 16, 16)] = dst_v[pl.ds(base + k * 16, 16)]
            pltpu.async_copy(rel_h.at[comp], rows_v, sem).wait()
            pltpu.sync_copy(rows_v, acc.at[dstb], add=True)
            return carry

        lax.fori_loop(0, nblk, blk, 0)

        plsc.subcore_barrier()

        for q in range(nrc):
            r0 = t * rpt + q * RC
            pltpu.sync_copy(acc.at[pl.ds(r0, RC)], rows_i)
            pltpu.sync_copy(rows_i, out_h.at[pl.ds(c * Nfp + r0, RC)])

    return functools.partial(
        pl.kernel,
        out_type=jax.ShapeDtypeStruct((2 * Nfp, 128), jnp.float32),
        mesh=_MESH,
        scratch_types=scratch,
    )(body)


def _make_tc_stage(Np, do_norm):
    nb = Np // 256

    def body(aggA, aggB, bpre, score, wrel, wroot, broot, rel_o, root_o, g):
        j = pl.program_id(1)

        @pl.when(j == 0)
        def _():
            a = jnp.concatenate([aggA[...], aggB[...]], axis=1)
            if do_norm:
                tfull = a + bpre[...]
                n = jnp.sqrt(jnp.sum(tfull * tfull, axis=1, keepdims=True))
                h = tfull / jnp.maximum(n, 1e-12)
                h = jnp.maximum(h, 0.0)
            else:
                h = a
            g[...] = h / score[...]

        gv = g[...]
        rel_o[...] = jnp.dot(gv, wrel[...], preferred_element_type=jnp.float32)
        root_o[...] = (
            jnp.dot(gv, wroot[...], preferred_element_type=jnp.float32)
            + broot[...]
        )

    out_sd = jax.ShapeDtypeStruct((2 * Np, 128), jnp.float32)
    return pl.pallas_call(
        body,
        grid=(nb, 2),
        in_specs=[
            pl.BlockSpec((256, 128), lambda i, j: (i, 0)),
            pl.BlockSpec((256, 128), lambda i, j: (i + nb, 0)),
            pl.BlockSpec((1, 256), lambda i, j: (0, 0)),
            pl.BlockSpec((256, 1), lambda i, j: (i, 0)),
            pl.BlockSpec((256, 128), lambda i, j: (0, j)),
            pl.BlockSpec((256, 128), lambda i, j: (0, 2 + j)),
            pl.BlockSpec((1, 128), lambda i, j: (0, j)),
        ],
        out_specs=[
            pl.BlockSpec((256, 128), lambda i, j: (j * nb + i, 0)),
            pl.BlockSpec((256, 128), lambda i, j: (j * nb + i, 0)),
        ],
        out_shape=[out_sd, out_sd],
        scratch_shapes=[pltpu.VMEM((256, 256), jnp.float32)],
    )


_sc_conv1 = _make_sc_conv(N2P, N1P, E1P, identity_cluster=False)
_sc_conv2 = _make_sc_conv(N1P, N0P, E0P, identity_cluster=False)
_sc_conv3 = _make_sc_conv(N0P, N0P, E0P, identity_cluster=True)
_tc_stage1 = _make_tc_stage(N2P, do_norm=False)
_tc_stage2 = _make_tc_stage(N1P, do_norm=True)
_tc_stage3 = _make_tc_stage(N0P, do_norm=True)


def _half_split(a, n):
    return a.reshape(n, 2, 128).transpose(1, 0, 2).reshape(2 * n, 128)


def _pad_edges(ei, e, ep, nf, nfp):
    pad = ep - e
    padi = jnp.arange(pad, dtype=jnp.int32)
    src = jnp.concatenate([ei[0], padi % nf])
    dst = jnp.concatenate([ei[1], nf + padi % (nfp - nf)])
    return src.reshape(16, ep // 16), dst.reshape(16, ep // 16)


def kernel(x, edge_index, cluster0, new_edge_score0, edge_index0, batch0,
           cluster1, new_edge_score1, edge_index1, batch1,
           W_rel0, b_rel0, W_root0, W_rel1, b_rel1, W_root1,
           W_rel2, b_rel2, W_root2):
    f32 = jnp.float32

    xflat = _half_split(jnp.pad(x, ((0, N2P - N2), (0, 0))), N2P)
    s1 = jnp.pad(new_edge_score1, (0, N2P - N2), constant_values=1.0)
    s1 = s1.reshape(N2P, 1)
    s0 = jnp.pad(new_edge_score0, (0, N1P - N1), constant_values=1.0)
    s0 = s0.reshape(N1P, 1)
    ones0 = jnp.ones((N0P, 1), f32)
    zb = jnp.zeros((1, H), f32)

    Wcat2 = jnp.concatenate([W_rel2.T, W_root2.T], axis=1)
    Wcat1 = jnp.concatenate([W_rel1.T, W_root1.T], axis=1)
    Wcat0 = jnp.concatenate([W_rel0.T, W_root0.T], axis=1)

    cl1 = jnp.pad(cluster1, (0, N1P - N1))
    cl0 = jnp.pad(cluster0, (0, N0P - N0))
    src1, dst1 = _pad_edges(edge_index1, E1, E1P, N1, N1P)
    src0, dst0 = _pad_edges(edge_index0, E0, E0P, N0, N0P)

    rel2, root2 = _tc_stage1(xflat, xflat, zb, s1, Wcat2, Wcat2, zb)
    agg1 = _sc_conv1(src1, dst1, cl1, rel2, root2)

    rel1, root1 = _tc_stage2(agg1, agg1, b_rel2.reshape(1, H), s0,
                             Wcat1, Wcat1, zb)
    agg0 = _sc_conv2(src0, dst0, cl0, rel1, root1)

    rel0, root0 = _tc_stage3(agg0, agg0, b_rel1.reshape(1, H), ones0,
                             Wcat0, Wcat0, b_rel0.reshape(1, H))
    out = _sc_conv3(src0, dst0, rel0, root0)

    return out.reshape(2, N0P, 128).transpose(1, 0, 2).reshape(N0P, H)[:N0]


# speedup vs baseline: 2.8878x; 2.8878x over previous
import functools

import jax
import jax.numpy as jnp
from jax import lax
from jax.experimental import pallas as pl
from jax.experimental.pallas import tpu as pltpu
from jax.experimental.pallas import tpu_sc as plsc

N0, N1, N2 = 10000, 5000, 2500
E0, E1 = 160000, 80000
H = 256

N0P, N1P, N2P = 10240, 5120, 2560
E0P, E1P = 163840, 81920

_MESH = plsc.VectorSubcoreMesh(core_axis_name="c", subcore_axis_name="s",
                               num_cores=2, num_subcores=16)


def _make_sc_conv(Ncp, Nfp, Ep, identity_cluster, n_chunks=2):
    CW = 256 // n_chunks
    cpc = n_chunks // 2
    per_tile = Ep // 16
    nblk = per_tile // 128
    rpt = Nfp // 16
    RC = 128 if rpt % 128 == 0 else 64
    nrc = rpt // RC

    scratch = []
    if not identity_cluster:
        scratch.append(pltpu.VMEM((Nfp,), jnp.int32))
    scratch += [
        pltpu.VMEM((per_tile,), jnp.int32),
        pltpu.VMEM((per_tile,), jnp.int32),
        pltpu.VMEM((128,), jnp.int32),
        pltpu.VMEM((128,), jnp.int32),
        pltpu.VMEM((RC,), jnp.int32),
        pltpu.VMEM((128, CW), jnp.float32),
        pltpu.VMEM((RC, CW), jnp.float32),
        pltpu.VMEM_SHARED((Nfp, CW), jnp.float32),
        pltpu.SemaphoreType.DMA,
    ]

    def body(*refs):
        if identity_cluster:
            (src_h, dst_h, rel_h, root_h, out_h,
             src_v, dst_v, comp, dstb, compi, rows_v, rows_i, acc, sem) = refs
            cluster_v = None
        else:
            (src_h, dst_h, cluster_h, rel_h, root_h, out_h,
             cluster_v, src_v, dst_v, comp, dstb, compi, rows_v, rows_i,
             acc, sem) = refs

        c = lax.axis_index("c")
        t = lax.axis_index("s")

        pltpu.sync_copy(src_h.at[t], src_v)
        pltpu.sync_copy(dst_h.at[t], dst_v)
        if not identity_cluster:
            pltpu.sync_copy(cluster_h, cluster_v)

        for cc in range(cpc):
            if n_chunks == 2:
                scale = 1
                t_off = (c * Ncp).astype(jnp.int32)
            else:
                scale = 2
                t_off = (2 * c * Ncp + cc).astype(jnp.int32)

            for q in range(nrc):
                r0 = t * rpt + q * RC
                if identity_cluster:
                    if n_chunks == 2:
                        pltpu.sync_copy(
                            root_h.at[pl.ds(c * Nfp + r0, RC)], rows_i)
                    else:
                        pltpu.sync_copy(
                            root_h.at[pl.ds(c * Nfp + r0, RC), cc], rows_i)
                else:
                    for k in range(RC // 16):
                        cv = cluster_v[pl.ds(r0 + k * 16, 16)]
                        compi[pl.ds(k * 16, 16)] = cv * scale + t_off
                    pltpu.async_copy(root_h.at[compi], rows_i, sem).wait()
                pltpu.sync_copy(rows_i, acc.at[pl.ds(r0, RC)])

            plsc.subcore_barrier()

            def blk(j, carry):
                base = j * 128
                for k in range(8):
                    s16 = src_v[pl.ds(base + k * 16, 16)]
                    if identity_cluster:
                        cg = s16
                    else:
                        cg = plsc.load_gather(cluster_v, [s16])
                    comp[pl.ds(k * 16, 16)] = cg * scale + t_off
                    dstb[pl.ds(k * 16, 16)] = dst_v[pl.ds(base + k * 16, 16)]
                pltpu.async_copy(rel_h.at[comp], rows_v, sem).wait()
                pltpu.sync_copy(rows_v, acc.at[dstb], add=True)
                return carry

            lax.fori_loop(0, nblk, blk, 0)

            plsc.subcore_barrier()

            for q in range(nrc):
                r0 = t * rpt + q * RC
                pltpu.sync_copy(acc.at[pl.ds(r0, RC)], rows_i)
                if n_chunks == 2:
                    pltpu.sync_copy(rows_i, out_h.at[pl.ds(c * Nfp + r0, RC)])
                else:
                    pltpu.sync_copy(
                        rows_i, out_h.at[pl.ds(c * Nfp + r0, RC), cc])

    if n_chunks == 2:
        out_sd = jax.ShapeDtypeStruct((2 * Nfp, 128), jnp.float32)
    else:
        out_sd = jax.ShapeDtypeStruct((2 * Nfp, 2, 64), jnp.float32)
    return functools.partial(
        pl.kernel,
        out_type=out_sd,
        mesh=_MESH,
        scratch_types=scratch,
        compiler_params=pltpu.CompilerParams(
            needs_layout_passes=False,
            use_tc_tiling_on_sc=False if n_chunks == 4 else None,
        ),
    )(body)


def _make_tc_stage(Np, do_norm):
    nb = Np // 256

    def body(aggA, aggB, bpre, score, wrel, wroot, broot, rel_o, root_o, g):
        j = pl.program_id(1)

        @pl.when(j == 0)
        def _():
            a = jnp.concatenate([aggA[...], aggB[...]], axis=1)
            if do_norm:
                tfull = a + bpre[...]
                n = jnp.sqrt(jnp.sum(tfull * tfull, axis=1, keepdims=True))
                h = tfull / jnp.maximum(n, 1e-12)
                h = jnp.maximum(h, 0.0)
            else:
                h = a
            g[...] = h / score[...]

        gv = g[...]
        rel_o[...] = jnp.dot(gv, wrel[...], preferred_element_type=jnp.float32)
        root_o[...] = (
            jnp.dot(gv, wroot[...], preferred_element_type=jnp.float32)
            + broot[...]
        )

    out_sd = jax.ShapeDtypeStruct((2 * Np, 128), jnp.float32)
    return pl.pallas_call(
        body,
        grid=(nb, 2),
        in_specs=[
            pl.BlockSpec((256, 128), lambda i, j: (i, 0)),
            pl.BlockSpec((256, 128), lambda i, j: (i + nb, 0)),
            pl.BlockSpec((1, 256), lambda i, j: (0, 0)),
            pl.BlockSpec((256, 1), lambda i, j: (i, 0)),
            pl.BlockSpec((256, 128), lambda i, j: (0, j)),
            pl.BlockSpec((256, 128), lambda i, j: (0, 2 + j)),
            pl.BlockSpec((1, 128), lambda i, j: (0, j)),
        ],
        out_specs=[
            pl.BlockSpec((256, 128), lambda i, j: (j * nb + i, 0)),
            pl.BlockSpec((256, 128), lambda i, j: (j * nb + i, 0)),
        ],
        out_shape=[out_sd, out_sd],
        scratch_shapes=[pltpu.VMEM((256, 256), jnp.float32)],
    )


_sc_conv1 = _make_sc_conv(N2P, N1P, E1P, identity_cluster=False, n_chunks=2)
_sc_conv2 = _make_sc_conv(N1P, N0P, E0P, identity_cluster=False, n_chunks=4)
_sc_conv3 = _make_sc_conv(N0P, N0P, E0P, identity_cluster=True, n_chunks=4)
_tc_stage1 = _make_tc_stage(N2P, do_norm=False)
_tc_stage2 = _make_tc_stage(N1P, do_norm=True)
_tc_stage3 = _make_tc_stage(N0P, do_norm=True)


def _half_split(a, n):
    return a.reshape(n, 2, 128).transpose(1, 0, 2).reshape(2 * n, 128)


def _pad_edges(ei, e, ep, nf, nfp):
    pad = ep - e
    padi = jnp.arange(pad, dtype=jnp.int32)
    src = jnp.concatenate([ei[0], padi % nf])
    dst = jnp.concatenate([ei[1], nf + padi % (nfp - nf)])
    return src.reshape(16, ep // 16), dst.reshape(16, ep // 16)


def kernel(x, edge_index, cluster0, new_edge_score0, edge_index0, batch0,
           cluster1, new_edge_score1, edge_index1, batch1,
           W_rel0, b_rel0, W_root0, W_rel1, b_rel1, W_root1,
           W_rel2, b_rel2, W_root2):
    f32 = jnp.float32

    xflat = _half_split(jnp.pad(x, ((0, N2P - N2), (0, 0))), N2P)
    s1 = jnp.pad(new_edge_score1, (0, N2P - N2), constant_values=1.0)
    s1 = s1.reshape(N2P, 1)
    s0 = jnp.pad(new_edge_score0, (0, N1P - N1), constant_values=1.0)
    s0 = s0.reshape(N1P, 1)
    ones0 = jnp.ones((N0P, 1), f32)
    zb = jnp.zeros((1, H), f32)

    Wcat2 = jnp.concatenate([W_rel2.T, W_root2.T], axis=1)
    Wcat1 = jnp.concatenate([W_rel1.T, W_root1.T], axis=1)
    Wcat0 = jnp.concatenate([W_rel0.T, W_root0.T], axis=1)

    cl1 = jnp.pad(cluster1, (0, N1P - N1))
    cl0 = jnp.pad(cluster0, (0, N0P - N0))
    src1, dst1 = _pad_edges(edge_index1, E1, E1P, N1, N1P)
    src0, dst0 = _pad_edges(edge_index0, E0, E0P, N0, N0P)

    rel2, root2 = _tc_stage1(xflat, xflat, zb, s1, Wcat2, Wcat2, zb)
    agg1 = _sc_conv1(src1, dst1, cl1, rel2, root2)

    rel1, root1 = _tc_stage2(agg1, agg1, b_rel2.reshape(1, H), s0,
                             Wcat1, Wcat1, zb)
    agg0 = _sc_conv2(src0, dst0, cl0,
                     rel1.reshape(4 * N1P, 64), root1.reshape(4 * N1P, 64))
    agg0 = agg0.reshape(2 * N0P, 128)

    rel0, root0 = _tc_stage3(agg0, agg0, b_rel1.reshape(1, H), ones0,
                             Wcat0, Wcat0, b_rel0.reshape(1, H))
    out = _sc_conv3(src0, dst0,
                    rel0.reshape(4 * N0P, 64), root0.reshape(2 * N0P, 2, 64))

    return (out.reshape(2, N0P, 128).transpose(1, 0, 2)
            .reshape(N0P, H)[:N0])
